# baseline (speedup 1.0000x reference)
import jax
import jax.numpy as jnp
from jax.experimental import pallas as pl
from jax.experimental.pallas import tpu as pltpu


def _select_kernel(ab_ref, img_ref, mask_ref, boost_ref, out_ref):
    a = ab_ref[0]
    b = ab_ref[1]
    _, ch, w = img_ref.shape
    h = mask_ref.shape[1]
    m = mask_ref[...].reshape(1, 1, h, w)
    img = img_ref[...].reshape(1, ch // h, h, w)
    bt = boost_ref[...].reshape(1, ch // h, h, w) * a + b
    out = jnp.where(m < 0.5, bt, img)
    out_ref[...] = out.reshape(1, ch, w)


def kernel(standardization, batchimg, batchmask, boost):
    batchimg = batchimg.astype(jnp.float32)
    batchmask = batchmask.astype(jnp.float32)
    boost = boost.astype(jnp.float32)
    B, C, H, W = batchimg.shape
    std = jnp.asarray(standardization)
    a = jnp.where(std != 0, jnp.float32(1.0 / 3.9), jnp.float32(0.5 / 3.9))
    b = jnp.where(std != 0, jnp.float32(0.0), jnp.float32(0.5))
    ab = jnp.stack([a, b]).astype(jnp.float32)

    out = pl.pallas_call(
        _select_kernel,
        grid=(B,),
        compiler_params=pltpu.CompilerParams(
            dimension_semantics=("arbitrary",),
        ),
        in_specs=[
            pl.BlockSpec(memory_space=pltpu.SMEM),
            pl.BlockSpec((1, C * H, W), lambda i: (i, 0, 0)),
            pl.BlockSpec((1, H, W), lambda i: (i, 0, 0)),
            pl.BlockSpec((1, C * H, W), lambda i: (i, 0, 0)),
        ],
        out_specs=pl.BlockSpec((1, C * H, W), lambda i: (i, 0, 0)),
        out_shape=jax.ShapeDtypeStruct((B, C * H, W), jnp.float32),
    )(ab, batchimg.reshape(B, C * H, W), batchmask,
      boost.reshape(B, C * H, W))
    return out.reshape(B, C, H, W)
